# baseline (speedup 1.0000x reference)
import jax
import jax.numpy as jnp
from jax import lax
from jax.experimental import pallas as pl
from jax.experimental.pallas import tpu as pltpu
from jax.experimental.pallas import tpu_sc as plsc

_NUM_CLASSES = 1000000
_FEAT_DIM = 64
_BATCH = 16384
_LAMBDA_C = 1.0

_NC = 2
_NS = 16
_NW = _NC * _NS
_LANES = 16
_SB = 512
_TAIL = (_NUM_CLASSES // _SB) * _SB
_NSB = _TAIL // _SB
_SB_PER_W = _NSB // _NW
_MAXPAIR = (_SB_PER_W + 2) // 2
_SCAN_GROUPS = _BATCH // _LANES
_STAGE = 16


def _k1_body(lab_hbm, centT_hbm, e_hbm,
             lab_all, blk_a, blk_b, srt, estage, acc_pad,
             cnt_s, off_s, cur_s, misc_s,
             sem_a, sem_b, sem_e):
    wid = lax.axis_index("s") * _NC + lax.axis_index("c")
    nsb = _SB_PER_W + jnp.where(wid == _NW - 1, 1, 0)
    lo = wid * (_SB_PER_W * _SB)
    rng = nsb * _SB
    iota = lax.iota(jnp.int32, _LANES)

    pltpu.sync_copy(lab_hbm, lab_all)

    def init_cnt(b, c):
        cnt_s[b] = 0
        return c
    lax.fori_loop(0, _SB_PER_W + 2, init_cnt, 0)

    def scan1(g, c):
        vec = lab_all[pl.ds(g * _LANES, _LANES)]
        rel = vec - lo
        mi = jnp.where((rel >= 0) & (rel < rng), 1, 0)
        npos = plsc.all_reduce_population_count(mi == 1)

        @pl.when(npos[0] > 0)
        def _():
            for l in range(_LANES):
                @pl.when(mi[l] == 1)
                def _():
                    sb = rel[l] >> 9
                    cnt_s[sb] = cnt_s[sb] + 1
        return c
    lax.fori_loop(0, _SCAN_GROUPS, scan1, 0)

    misc_s[0] = 0

    def prefix(b, c):
        v = misc_s[0]
        off_s[b] = v
        cur_s[b] = v
        misc_s[0] = v + cnt_s[b]
        return c
    lax.fori_loop(0, _SB_PER_W + 2, prefix, 0)

    lane0 = iota == 0

    def scan2(g, c):
        vec = lab_all[pl.ds(g * _LANES, _LANES)]
        rel = vec - lo
        mi = jnp.where((rel >= 0) & (rel < rng), 1, 0)
        npos = plsc.all_reduce_population_count(mi == 1)

        @pl.when(npos[0] > 0)
        def _():
            for l in range(_LANES):
                @pl.when(mi[l] == 1)
                def _():
                    sb = rel[l] >> 9
                    p = cur_s[sb]
                    cur_s[sb] = p + 1
                    val = rel[l] * _BATCH + g * _LANES + l
                    plsc.store_scatter(
                        srt, [jnp.full((_LANES,), p, jnp.int32)],
                        jnp.full((_LANES,), val, jnp.int32), mask=lane0)
        return c
    lax.fori_loop(0, _SCAN_GROUPS, scan2, 0)

    misc_s[1] = 0

    def fire(sb_local, buf, sem):
        return pltpu.async_copy(
            centT_hbm.at[:, pl.ds(lo + sb_local * _SB, _SB)], buf, sem)

    @pl.when(0 < nsb)
    def _():
        fire(0, blk_a, sem_a)

    @pl.when(1 < nsb)
    def _():
        fire(1, blk_b, sem_b)

    def extract(sb, buf):
        m0 = off_s[sb]
        m1 = off_s[sb + 1]

        def per_match(m, c, buf=buf, sb=sb):
            v = plsc.load_gather(srt, [jnp.full((_LANES,), m, jnp.int32)])
            val = v[0]
            rel = val // _BATCH
            b = val - rel * _BATCH
            col = rel & (_SB - 1)
            colv = jnp.full((_LANES,), col, jnp.int32)
            rc = misc_s[1]
            slot = rc & (_STAGE - 1)

            @pl.when(rc >= _STAGE)
            def _():
                pltpu.make_async_copy(
                    e_hbm.at[pl.ds(0, 1), :],
                    estage.at[pl.ds(_STAGE, 1), :], sem_e).wait()

            for gg in range(_FEAT_DIM // _LANES):
                dvec = iota + gg * _LANES
                vals = plsc.load_gather(buf, [dvec, colv])
                estage[slot, pl.ds(gg * _LANES, _LANES)] = vals
            pltpu.async_copy(estage.at[pl.ds(slot, 1), :],
                             e_hbm.at[pl.ds(b, 1), :], sem_e)
            misc_s[1] = rc + 1
            return c

        lax.fori_loop(m0, m1, per_match, 0)

    def pair_step(tp, c):
        sb0 = tp * 2
        sb1 = sb0 + 1

        @pl.when(sb0 < nsb)
        def _():
            pltpu.make_async_copy(
                centT_hbm.at[:, pl.ds(0, _SB)], blk_a, sem_a).wait()
            extract(sb0, blk_a)

            @pl.when(sb0 + 2 < nsb)
            def _():
                fire(sb0 + 2, blk_a, sem_a)

        @pl.when(sb1 < nsb)
        def _():
            pltpu.make_async_copy(
                centT_hbm.at[:, pl.ds(0, _SB)], blk_b, sem_b).wait()
            extract(sb1, blk_b)

            @pl.when(sb1 + 2 < nsb)
            def _():
                fire(sb1 + 2, blk_b, sem_b)
        return c

    lax.fori_loop(0, _MAXPAIR, pair_step, 0)

    def drain(i, c):
        pltpu.make_async_copy(
            e_hbm.at[pl.ds(0, 1), :],
            estage.at[pl.ds(_STAGE, 1), :], sem_e).wait()
        return c
    lax.fori_loop(0, jnp.minimum(misc_s[1], _STAGE), drain, 0)


def _k1(labels, centersT):
    mesh = plsc.VectorSubcoreMesh(core_axis_name="c", subcore_axis_name="s")
    return pl.kernel(
        _k1_body,
        mesh=mesh,
        compiler_params=pltpu.CompilerParams(needs_layout_passes=False),
        out_type=jax.ShapeDtypeStruct((_BATCH, _FEAT_DIM), jnp.float32),
        scratch_types=[
            pltpu.VMEM((_BATCH,), jnp.int32),
            pltpu.VMEM((_FEAT_DIM, _SB), jnp.float32),
            pltpu.VMEM((_FEAT_DIM, _SB), jnp.float32),
            pltpu.VMEM((_BATCH,), jnp.int32),
            pltpu.VMEM((_STAGE + 1, _FEAT_DIM), jnp.float32),
            pltpu.VMEM((_LANES,), jnp.float32),
            pltpu.SMEM((_SB_PER_W + 2,), jnp.int32),
            pltpu.SMEM((_SB_PER_W + 2,), jnp.int32),
            pltpu.SMEM((_SB_PER_W + 2,), jnp.int32),
            pltpu.SMEM((8,), jnp.int32),
            pltpu.SemaphoreType.DMA,
            pltpu.SemaphoreType.DMA,
            pltpu.SemaphoreType.DMA,
        ],
    )(labels, centersT)


_BLK = 1024
_GRID = _BATCH // _BLK


def _k2_body(featT_ref, e_ref, lab_ref, tail_ref, out_ref):
    i = pl.program_id(0)

    @pl.when(i == 0)
    def _():
        out_ref[0, 0] = 0.0

    ft = featT_ref[...]
    e = e_ref[...]
    lab = lab_ref[...]
    tail = tail_ref[...]

    is_tail = lab >= _TAIL
    rowids = lax.broadcasted_iota(jnp.int32, (1, _FEAT_DIM), 1) + _TAIL
    oh = jnp.where(lab == rowids, 1.0, 0.0)
    texp = jax.lax.dot(oh, tail, precision=jax.lax.Precision.HIGHEST)
    e_sel = jnp.where(is_tail, texp, e)

    m = jax.lax.dot(ft, e_sel, precision=jax.lax.Precision.HIGHEST)
    eye = jnp.where(
        lax.broadcasted_iota(jnp.int32, (_FEAT_DIM, _FEAT_DIM), 0)
        == lax.broadcasted_iota(jnp.int32, (_FEAT_DIM, _FEAT_DIM), 1),
        1.0, 0.0)
    cross = jnp.sum(m * eye)
    total = jnp.sum(ft * ft) + jnp.sum(e_sel * e_sel) - 2.0 * cross
    out_ref[0, 0] += total * (_LAMBDA_C / float(_BATCH * _FEAT_DIM))


def _k2(featT, e, labels2d, tail):
    return pl.pallas_call(
        _k2_body,
        grid=(_GRID,),
        in_specs=[
            pl.BlockSpec((_FEAT_DIM, _BLK), lambda i: (0, i)),
            pl.BlockSpec((_BLK, _FEAT_DIM), lambda i: (i, 0)),
            pl.BlockSpec((_BLK, 1), lambda i: (i, 0)),
            pl.BlockSpec((_FEAT_DIM, _FEAT_DIM), lambda i: (0, 0)),
        ],
        out_specs=pl.BlockSpec(memory_space=pltpu.SMEM),
        out_shape=jax.ShapeDtypeStruct((1, 1), jnp.float32),
    )(featT, e, labels2d, tail)


@jax.jit
def kernel(features, labels, centers):
    lab = labels.astype(jnp.int32)
    centersT = centers.T
    e = _k1(lab, centersT)
    tail = lax.slice(centersT, (0, _TAIL), (_FEAT_DIM, _NUM_CLASSES))
    tail = jnp.transpose(tail)
    loss = _k2(features.T, e, lab.reshape(_BATCH, 1), tail)
    return loss[0, 0]
